# baseline (speedup 1.0000x reference)
import jax
import jax.numpy as jnp
from jax import lax
from jax.experimental import pallas as pl
from jax.experimental.pallas import tpu as pltpu
from jax.experimental.pallas import tpu_sc as plsc

_N = 10000
_E = 320000
_D = 128
_F = 64

_NC = 2
_NS = 16
_NW = _NC * _NS
_CHUNK = 128
_NCHUNK = _E // _CHUNK
_CBASE = _NCHUNK // _NW
_CREM = _NCHUNK % _NW
_CMAX = _CBASE + 1
_NPAD = 10240
_RPT = _NPAD // _NS
_NBUF = 4

_mesh = plsc.VectorSubcoreMesh(core_axis_name="c", subcore_axis_name="s")
_sc_params = pltpu.CompilerParams(use_tc_tiling_on_sc=False)


def _my_chunks(wid):
    lo = _NW - _CREM
    base = _CBASE * wid + jnp.maximum(wid - lo, 0)
    n = _CBASE + jnp.where(wid >= lo, 1, 0)
    return base, n


def _deg_body(edges_hbm, deg_out, dst_v, ones_v, zb_v, deg_sh, sem):
    c = lax.axis_index("c")
    s = lax.axis_index("s")
    wid = c * _NS + s
    base, n = _my_chunks(wid)
    cp = pltpu.async_copy(edges_hbm.at[1, pl.ds(base, _CMAX), :], dst_v, sem)
    for i in range(_CHUNK // 16):
        ones_v[pl.ds(i * 16, 16)] = jnp.ones((16,), jnp.float32)

    def zfill(i, carry):
        zb_v[pl.ds(i * 16, 16)] = jnp.zeros((16,), jnp.float32)
        return carry

    lax.fori_loop(0, _RPT // 16, zfill, 0)
    pltpu.sync_copy(zb_v, deg_sh.at[pl.ds(s * _RPT, _RPT)])
    cp.wait()
    plsc.subcore_barrier()

    def body(j, carry):
        pltpu.sync_copy(ones_v, deg_sh.at[dst_v.at[j]], add=True)
        return carry

    lax.fori_loop(0, n, body, 0)
    plsc.subcore_barrier()
    pltpu.sync_copy(deg_sh.at[pl.ds(s * _RPT, _RPT)], zb_v)
    pltpu.sync_copy(zb_v, deg_out.at[pl.ds(c * _NPAD + s * _RPT, _RPT)])


_deg_kernel = pl.kernel(
    _deg_body,
    out_type=jax.ShapeDtypeStruct((_NC * _NPAD,), jnp.float32),
    mesh=_mesh,
    compiler_params=_sc_params,
    scratch_types=[
        pltpu.VMEM((_CMAX, _CHUNK), jnp.int32),
        pltpu.VMEM((_CHUNK,), jnp.float32),
        pltpu.VMEM((_RPT,), jnp.float32),
        pltpu.VMEM_SHARED((_NPAD,), jnp.float32),
        pltpu.SemaphoreType.DMA,
    ],
)


def _agg_body(hs_hbm, edges_hbm, agg_out,
              src_v, dst_v, rows, acc_sh, gsems, ssems, sem):
    c = lax.axis_index("c")
    s = lax.axis_index("s")
    wid = c * _NS + s
    base, n = _my_chunks(wid)
    cp1 = pltpu.async_copy(edges_hbm.at[0, pl.ds(base, _CMAX), :], src_v, sem)
    cp2 = pltpu.async_copy(edges_hbm.at[1, pl.ds(base, _CMAX), :], dst_v, sem)

    def zfill(j, carry):
        for k in range(_F // 16):
            rows[0][j, pl.ds(k * 16, 16)] = jnp.zeros((16,), jnp.float32)
        return carry

    lax.fori_loop(0, _CHUNK, zfill, 0)
    for k in range(_RPT // _CHUNK):
        pltpu.sync_copy(rows[0],
                        acc_sh.at[pl.ds(s * _RPT + k * _CHUNK, _CHUNK), :])
    cp1.wait()
    cp2.wait()
    for k in range(_NBUF):
        pltpu.async_copy(hs_hbm.at[src_v.at[k]], rows[k], gsems[k])
    plsc.subcore_barrier()

    ngrp = n // _NBUF
    nrem = n - ngrp * _NBUF

    def body(g, carry):
        j = g * _NBUF
        for k in range(_NBUF):
            pltpu.make_async_copy(hs_hbm.at[src_v.at[j + k]],
                                  rows[k], gsems[k]).wait()
            pltpu.async_copy(rows[k], acc_sh.at[dst_v.at[j + k]], ssems[k],
                             add=True)
        for k in range(_NBUF):
            pltpu.make_async_copy(rows[k], acc_sh.at[dst_v.at[j + k]],
                                  ssems[k]).wait()
            jn = jnp.minimum(j + _NBUF + k, n - 1)
            pltpu.async_copy(hs_hbm.at[src_v.at[jn]], rows[k], gsems[k])
        return carry

    lax.fori_loop(0, ngrp, body, 0)

    def tail_k(k):
        j = ngrp * _NBUF + k
        pltpu.make_async_copy(hs_hbm.at[src_v.at[jnp.minimum(j, n - 1)]],
                              rows[k], gsems[k]).wait()

        @pl.when(k < nrem)
        def _():
            pltpu.sync_copy(rows[k], acc_sh.at[dst_v.at[j]], add=True)

    for k in range(_NBUF):
        tail_k(k)
    plsc.subcore_barrier()
    for k in range(_RPT // _CHUNK):
        pltpu.sync_copy(acc_sh.at[pl.ds(s * _RPT + k * _CHUNK, _CHUNK), :],
                        rows[0])
        pltpu.sync_copy(rows[0],
                        agg_out.at[c, pl.ds(s * _RPT + k * _CHUNK, _CHUNK), :])


_agg_kernel = pl.kernel(
    _agg_body,
    out_type=jax.ShapeDtypeStruct((_NC, _NPAD, _F), jnp.float32),
    mesh=_mesh,
    compiler_params=_sc_params,
    scratch_types=[
        pltpu.VMEM((_CMAX, _CHUNK), jnp.int32),
        pltpu.VMEM((_CMAX, _CHUNK), jnp.int32),
        [pltpu.VMEM((_CHUNK, _F), jnp.float32) for _ in range(_NBUF)],
        pltpu.VMEM_SHARED((_NPAD, _F), jnp.float32),
        [pltpu.SemaphoreType.DMA for _ in range(_NBUF)],
        [pltpu.SemaphoreType.DMA for _ in range(_NBUF)],
        pltpu.SemaphoreType.DMA,
    ],
)


_BM = 1024
_GRID = (_N + _BM - 1) // _BM


def _dis_col(deg_ref):
    deg = deg_ref[0, :, :] + deg_ref[1, :, :]
    dis = lax.rsqrt(jnp.maximum(deg, 1.0))
    rsel = (lax.broadcasted_iota(jnp.int32, (_BM, 8), 0) // 128 ==
            lax.broadcasted_iota(jnp.int32, (_BM, 8), 1)).astype(jnp.float32)
    expanded = jnp.dot(rsel, dis, preferred_element_type=jnp.float32,
                       precision=lax.Precision.HIGHEST)
    lsel = (lax.broadcasted_iota(jnp.int32, (_BM, 128), 0) % 128 ==
            lax.broadcasted_iota(jnp.int32, (_BM, 128), 1))
    return jnp.sum(jnp.where(lsel, expanded, 0.0), axis=1, keepdims=True)


def _mm_body(x_ref, w_ref, deg_ref, hs_ref):
    h = jnp.dot(x_ref[...], w_ref[...], preferred_element_type=jnp.float32)
    hs_ref[...] = h * _dis_col(deg_ref)


def _mm_call(x, W, deg3):
    return pl.pallas_call(
        _mm_body,
        grid=(_GRID,),
        in_specs=[
            pl.BlockSpec((_BM, _D), lambda i: (i, 0)),
            pl.BlockSpec((_D, _F), lambda i: (0, 0)),
            pl.BlockSpec((_NC, _BM // 128, 128), lambda i: (0, i, 0)),
        ],
        out_specs=pl.BlockSpec((_BM, _F), lambda i: (i, 0)),
        out_shape=jax.ShapeDtypeStruct((_N, _F), jnp.float32),
    )(x, W, deg3)


def _fin_body(agg_ref, deg_ref, b_ref, out_ref):
    agg = agg_ref[0, :, :] + agg_ref[1, :, :]
    out_ref[...] = agg * _dis_col(deg_ref) + b_ref[0, :]


def _fin_call(agg_parts, deg3, b2):
    return pl.pallas_call(
        _fin_body,
        grid=(_GRID,),
        in_specs=[
            pl.BlockSpec((_NC, _BM, _F), lambda i: (0, i, 0)),
            pl.BlockSpec((_NC, _BM // 128, 128), lambda i: (0, i, 0)),
            pl.BlockSpec((1, _F), lambda i: (0, 0)),
        ],
        out_specs=pl.BlockSpec((_BM, _F), lambda i: (i, 0)),
        out_shape=jax.ShapeDtypeStruct((_N, _F), jnp.float32),
    )(agg_parts, deg3, b2)


def kernel(x, edge_index, W, b):
    edges = edge_index.reshape(2, _NCHUNK, _CHUNK)
    deg_parts = _deg_kernel(edges)
    deg3 = deg_parts.reshape(_NC, _NPAD // 128, 128)
    hs = _mm_call(x, W, deg3)
    agg_parts = _agg_kernel(hs, edges)
    out = _fin_call(agg_parts, deg3, b.reshape(1, _F))
    return out


# speedup vs baseline: 1.0232x; 1.0232x over previous
import jax
import jax.numpy as jnp
from jax import lax
from jax.experimental import pallas as pl
from jax.experimental.pallas import tpu as pltpu
from jax.experimental.pallas import tpu_sc as plsc

_N = 10000
_E = 320000
_D = 128
_F = 64

_NC = 2
_NS = 16
_NW = _NC * _NS
_CHUNK = 128
_NCHUNK = _E // _CHUNK
_CBASE = _NCHUNK // _NW
_CREM = _NCHUNK % _NW
_CMAX = _CBASE + 1
_NPAD = 10240
_RPT = _NPAD // _NS
_NBUF = 4

_mesh = plsc.VectorSubcoreMesh(core_axis_name="c", subcore_axis_name="s")
_sc_params = pltpu.CompilerParams(use_tc_tiling_on_sc=False)


def _my_chunks(wid):
    lo = _NW - _CREM
    base = _CBASE * wid + jnp.maximum(wid - lo, 0)
    n = _CBASE + jnp.where(wid >= lo, 1, 0)
    return base, n


def _deg_body(edges_hbm, deg_out, dst_v, ones_v, zb_v, deg_sh, sem):
    c = lax.axis_index("c")
    s = lax.axis_index("s")
    wid = c * _NS + s
    base, n = _my_chunks(wid)
    cp = pltpu.async_copy(edges_hbm.at[1, pl.ds(base, _CMAX), :], dst_v, sem)
    for i in range(_CHUNK // 16):
        ones_v[pl.ds(i * 16, 16)] = jnp.ones((16,), jnp.float32)

    def zfill(i, carry):
        zb_v[pl.ds(i * 16, 16)] = jnp.zeros((16,), jnp.float32)
        return carry

    lax.fori_loop(0, _RPT // 16, zfill, 0)
    pltpu.sync_copy(zb_v, deg_sh.at[pl.ds(s * _RPT, _RPT)])
    cp.wait()
    plsc.subcore_barrier()

    def body(j, carry):
        pltpu.sync_copy(ones_v, deg_sh.at[dst_v.at[j]], add=True)
        return carry

    lax.fori_loop(0, n, body, 0)
    plsc.subcore_barrier()
    pltpu.sync_copy(deg_sh.at[pl.ds(s * _RPT, _RPT)], zb_v)
    pltpu.sync_copy(zb_v, deg_out.at[pl.ds(c * _NPAD + s * _RPT, _RPT)])


_deg_kernel = pl.kernel(
    _deg_body,
    out_type=jax.ShapeDtypeStruct((_NC * _NPAD,), jnp.float32),
    mesh=_mesh,
    compiler_params=_sc_params,
    scratch_types=[
        pltpu.VMEM((_CMAX, _CHUNK), jnp.int32),
        pltpu.VMEM((_CHUNK,), jnp.float32),
        pltpu.VMEM((_RPT,), jnp.float32),
        pltpu.VMEM_SHARED((_NPAD,), jnp.float32),
        pltpu.SemaphoreType.DMA,
    ],
)


def _agg_body(hs_hbm, edges_hbm, agg_out,
              src_v, dst_v, rows, acc_sh, gsems, ssems, sem):
    c = lax.axis_index("c")
    s = lax.axis_index("s")
    wid = c * _NS + s
    base, n = _my_chunks(wid)
    cp1 = pltpu.async_copy(edges_hbm.at[0, pl.ds(base, _CMAX), :], src_v, sem)
    cp2 = pltpu.async_copy(edges_hbm.at[1, pl.ds(base, _CMAX), :], dst_v, sem)

    def zfill(j, carry):
        for k in range(_F // 16):
            rows[0][j, pl.ds(k * 16, 16)] = jnp.zeros((16,), jnp.float32)
        return carry

    lax.fori_loop(0, _CHUNK, zfill, 0)
    for k in range(_RPT // _CHUNK):
        pltpu.sync_copy(rows[0],
                        acc_sh.at[pl.ds(s * _RPT + k * _CHUNK, _CHUNK), :])
    cp1.wait()
    cp2.wait()
    for k in range(_NBUF):
        pltpu.async_copy(hs_hbm.at[src_v.at[k]], rows[k], gsems[k])
    plsc.subcore_barrier()

    ngrp = n // _NBUF
    nrem = n - ngrp * _NBUF

    def body(g, carry):
        j = g * _NBUF
        for k in range(_NBUF):
            pltpu.make_async_copy(hs_hbm.at[src_v.at[j + k]],
                                  rows[k], gsems[k]).wait()
            pltpu.async_copy(rows[k], acc_sh.at[dst_v.at[j + k]], ssems[k],
                             add=True)
        for k in range(_NBUF):
            pltpu.make_async_copy(rows[k], acc_sh.at[dst_v.at[j + k]],
                                  ssems[k]).wait()
            jn = jnp.minimum(j + _NBUF + k, n - 1)
            pltpu.async_copy(hs_hbm.at[src_v.at[jn]], rows[k], gsems[k])
        return carry

    lax.fori_loop(0, ngrp, body, 0)

    def tail_k(k):
        j = ngrp * _NBUF + k
        pltpu.make_async_copy(hs_hbm.at[src_v.at[jnp.minimum(j, n - 1)]],
                              rows[k], gsems[k]).wait()

        @pl.when(k < nrem)
        def _():
            pltpu.sync_copy(rows[k], acc_sh.at[dst_v.at[j]], add=True)

    for k in range(_NBUF):
        tail_k(k)
    plsc.subcore_barrier()
    for k in range(_RPT // _CHUNK):
        pltpu.sync_copy(acc_sh.at[pl.ds(s * _RPT + k * _CHUNK, _CHUNK), :],
                        rows[0])
        pltpu.sync_copy(rows[0],
                        agg_out.at[c, pl.ds(s * _RPT + k * _CHUNK, _CHUNK), :])


_agg_kernel = pl.kernel(
    _agg_body,
    out_type=jax.ShapeDtypeStruct((_NC, _NPAD, _F), jnp.float32),
    mesh=_mesh,
    compiler_params=_sc_params,
    scratch_types=[
        pltpu.VMEM((_CMAX, _CHUNK), jnp.int32),
        pltpu.VMEM((_CMAX, _CHUNK), jnp.int32),
        [pltpu.VMEM((_CHUNK, _F), jnp.float32) for _ in range(_NBUF)],
        pltpu.VMEM_SHARED((_NPAD, _F), jnp.float32),
        [pltpu.SemaphoreType.DMA for _ in range(_NBUF)],
        [pltpu.SemaphoreType.DMA for _ in range(_NBUF)],
        pltpu.SemaphoreType.DMA,
    ],
)


_BM = 1024
_GRID = (_N + _BM - 1) // _BM


def _dis_col(deg_ref):
    deg = deg_ref[0, :, :] + deg_ref[1, :, :]
    dis = lax.rsqrt(jnp.maximum(deg, 1.0))
    dis_t = dis.T
    return jnp.concatenate([dis_t[:, r:r + 1] for r in range(8)], axis=0)


def _mm_body(x_ref, w_ref, deg_ref, hs_ref):
    h = jnp.dot(x_ref[...], w_ref[...], preferred_element_type=jnp.float32)
    hs_ref[...] = h * _dis_col(deg_ref)


def _mm_call(x, W, deg3):
    return pl.pallas_call(
        _mm_body,
        grid=(_GRID,),
        in_specs=[
            pl.BlockSpec((_BM, _D), lambda i: (i, 0)),
            pl.BlockSpec((_D, _F), lambda i: (0, 0)),
            pl.BlockSpec((_NC, _BM // 128, 128), lambda i: (0, i, 0)),
        ],
        out_specs=pl.BlockSpec((_BM, _F), lambda i: (i, 0)),
        out_shape=jax.ShapeDtypeStruct((_N, _F), jnp.float32),
    )(x, W, deg3)


def _fin_body(agg_ref, deg_ref, b_ref, out_ref):
    agg = agg_ref[0, :, :] + agg_ref[1, :, :]
    out_ref[...] = agg * _dis_col(deg_ref) + b_ref[0, :]


def _fin_call(agg_parts, deg3, b2):
    return pl.pallas_call(
        _fin_body,
        grid=(_GRID,),
        in_specs=[
            pl.BlockSpec((_NC, _BM, _F), lambda i: (0, i, 0)),
            pl.BlockSpec((_NC, _BM // 128, 128), lambda i: (0, i, 0)),
            pl.BlockSpec((1, _F), lambda i: (0, 0)),
        ],
        out_specs=pl.BlockSpec((_BM, _F), lambda i: (i, 0)),
        out_shape=jax.ShapeDtypeStruct((_N, _F), jnp.float32),
    )(agg_parts, deg3, b2)


def kernel(x, edge_index, W, b):
    edges = edge_index.reshape(2, _NCHUNK, _CHUNK)
    deg_parts = _deg_kernel(edges)
    deg3 = deg_parts.reshape(_NC, _NPAD // 128, 128)
    hs = _mm_call(x, W, deg3)
    agg_parts = _agg_kernel(hs, edges)
    out = _fin_call(agg_parts, deg3, b.reshape(1, _F))
    return out


# speedup vs baseline: 1.0571x; 1.0331x over previous
import jax
import jax.numpy as jnp
from jax import lax
from jax.experimental import pallas as pl
from jax.experimental.pallas import tpu as pltpu
from jax.experimental.pallas import tpu_sc as plsc

_N = 10000
_E = 320000
_D = 128
_F = 64

_NC = 2
_NS = 16
_NW = _NC * _NS
_CHUNK = 128
_NCHUNK = _E // _CHUNK
_CBASE = _NCHUNK // _NW
_CREM = _NCHUNK % _NW
_CMAX = _CBASE + 1
_NPAD = 10240
_RPT = _NPAD // _NS
_NBUF = 4

_mesh = plsc.VectorSubcoreMesh(core_axis_name="c", subcore_axis_name="s")
_sc_params = pltpu.CompilerParams(use_tc_tiling_on_sc=False)


def _my_chunks(wid):
    lo = _NW - _CREM
    base = _CBASE * wid + jnp.maximum(wid - lo, 0)
    n = _CBASE + jnp.where(wid >= lo, 1, 0)
    return base, n


def _deg_body(edges_hbm, deg_out, dst_v, ones_v, zb_v, deg_sh, sem):
    c = lax.axis_index("c")
    s = lax.axis_index("s")
    wid = c * _NS + s
    base, n = _my_chunks(wid)
    cp = pltpu.async_copy(edges_hbm.at[1, pl.ds(base, _CMAX), :], dst_v, sem)
    for i in range(_CHUNK // 16):
        ones_v[pl.ds(i * 16, 16)] = jnp.ones((16,), jnp.float32)

    def zfill(i, carry):
        zb_v[pl.ds(i * 16, 16)] = jnp.zeros((16,), jnp.float32)
        return carry

    lax.fori_loop(0, _RPT // 16, zfill, 0)
    pltpu.sync_copy(zb_v, deg_sh.at[pl.ds(s * _RPT, _RPT)])
    cp.wait()
    plsc.subcore_barrier()

    def body(j, carry):
        pltpu.sync_copy(ones_v, deg_sh.at[dst_v.at[j]], add=True)
        return carry

    lax.fori_loop(0, n, body, 0)
    plsc.subcore_barrier()
    pltpu.sync_copy(deg_sh.at[pl.ds(s * _RPT, _RPT)], zb_v)
    pltpu.sync_copy(zb_v, deg_out.at[pl.ds(c * _NPAD + s * _RPT, _RPT)])


_deg_kernel = pl.kernel(
    _deg_body,
    out_type=jax.ShapeDtypeStruct((_NC * _NPAD,), jnp.float32),
    mesh=_mesh,
    compiler_params=_sc_params,
    scratch_types=[
        pltpu.VMEM((_CMAX, _CHUNK), jnp.int32),
        pltpu.VMEM((_CHUNK,), jnp.float32),
        pltpu.VMEM((_RPT,), jnp.float32),
        pltpu.VMEM_SHARED((_NPAD,), jnp.float32),
        pltpu.SemaphoreType.DMA,
    ],
)


def _agg_body(hs_hbm, edges_hbm, agg_out,
              src_v, dst_v, rows, acc_sh, gsems, ssems, sem):
    c = lax.axis_index("c")
    s = lax.axis_index("s")
    wid = c * _NS + s
    base, n = _my_chunks(wid)
    cp1 = pltpu.async_copy(edges_hbm.at[0, pl.ds(base, _CMAX), :], src_v, sem)
    cp2 = pltpu.async_copy(edges_hbm.at[1, pl.ds(base, _CMAX), :], dst_v, sem)

    def zfill(j, carry):
        for k in range(_F // 16):
            rows[0][j, pl.ds(k * 16, 16)] = jnp.zeros((16,), jnp.float32)
        return carry

    lax.fori_loop(0, _CHUNK, zfill, 0)
    for k in range(_RPT // _CHUNK):
        pltpu.sync_copy(rows[0],
                        acc_sh.at[pl.ds(s * _RPT + k * _CHUNK, _CHUNK), :])
    cp1.wait()
    cp2.wait()
    for k in range(_NBUF):
        pltpu.async_copy(hs_hbm.at[src_v.at[k]], rows[k], gsems[k])
    plsc.subcore_barrier()

    ngrp = n // _NBUF
    nrem = n - ngrp * _NBUF

    def body(g, carry):
        j = g * _NBUF
        for k in range(_NBUF):
            pltpu.make_async_copy(hs_hbm.at[src_v.at[j + k]],
                                  rows[k], gsems[k]).wait()
            pltpu.async_copy(rows[k], acc_sh.at[dst_v.at[j + k]], ssems[k],
                             add=True)
        for k in range(_NBUF):
            pltpu.make_async_copy(rows[k], acc_sh.at[dst_v.at[j + k]],
                                  ssems[k]).wait()
            jn = jnp.minimum(j + _NBUF + k, n - 1)
            pltpu.async_copy(hs_hbm.at[src_v.at[jn]], rows[k], gsems[k])
        return carry

    lax.fori_loop(0, ngrp, body, 0)

    def tail_k(k):
        j = ngrp * _NBUF + k
        pltpu.make_async_copy(hs_hbm.at[src_v.at[jnp.minimum(j, n - 1)]],
                              rows[k], gsems[k]).wait()

        @pl.when(k < nrem)
        def _():
            pltpu.sync_copy(rows[k], acc_sh.at[dst_v.at[j]], add=True)

    for k in range(_NBUF):
        tail_k(k)
    plsc.subcore_barrier()
    for k in range(_RPT // _CHUNK):
        pltpu.sync_copy(acc_sh.at[pl.ds(s * _RPT + k * _CHUNK, _CHUNK), :],
                        rows[0])
        pltpu.sync_copy(rows[0],
                        agg_out.at[c, pl.ds(s * _RPT + k * _CHUNK, _CHUNK), :])


_agg_kernel = pl.kernel(
    _agg_body,
    out_type=jax.ShapeDtypeStruct((_NC, _NPAD, _F), jnp.float32),
    mesh=_mesh,
    compiler_params=_sc_params,
    scratch_types=[
        pltpu.VMEM((_CMAX, _CHUNK), jnp.int32),
        pltpu.VMEM((_CMAX, _CHUNK), jnp.int32),
        [pltpu.VMEM((_CHUNK, _F), jnp.float32) for _ in range(_NBUF)],
        pltpu.VMEM_SHARED((_NPAD, _F), jnp.float32),
        [pltpu.SemaphoreType.DMA for _ in range(_NBUF)],
        [pltpu.SemaphoreType.DMA for _ in range(_NBUF)],
        pltpu.SemaphoreType.DMA,
    ],
)


_BM = 2048
_GRID = (_N + _BM - 1) // _BM


def _dis_col(deg_ref):
    deg = deg_ref[0, :, :] + deg_ref[1, :, :]
    dis = lax.rsqrt(jnp.maximum(deg, 1.0))
    dis_t = dis.T
    return jnp.concatenate(
        [dis_t[:, r:r + 1] for r in range(_BM // 128)], axis=0)


def _mm_body(x_ref, w_ref, deg_ref, hs_ref):
    h = jnp.dot(x_ref[...], w_ref[...], preferred_element_type=jnp.float32)
    hs_ref[...] = h * _dis_col(deg_ref)


def _mm_call(x, W, deg3):
    return pl.pallas_call(
        _mm_body,
        grid=(_GRID,),
        in_specs=[
            pl.BlockSpec((_BM, _D), lambda i: (i, 0)),
            pl.BlockSpec((_D, _F), lambda i: (0, 0)),
            pl.BlockSpec((_NC, _BM // 128, 128), lambda i: (0, i, 0)),
        ],
        out_specs=pl.BlockSpec((_BM, _F), lambda i: (i, 0)),
        out_shape=jax.ShapeDtypeStruct((_N, _F), jnp.float32),
    )(x, W, deg3)


def _fin_body(agg_ref, deg_ref, b_ref, out_ref):
    agg = agg_ref[0, :, :] + agg_ref[1, :, :]
    out_ref[...] = agg * _dis_col(deg_ref) + b_ref[0, :]


def _fin_call(agg_parts, deg3, b2):
    return pl.pallas_call(
        _fin_body,
        grid=(_GRID,),
        in_specs=[
            pl.BlockSpec((_NC, _BM, _F), lambda i: (0, i, 0)),
            pl.BlockSpec((_NC, _BM // 128, 128), lambda i: (0, i, 0)),
            pl.BlockSpec((1, _F), lambda i: (0, 0)),
        ],
        out_specs=pl.BlockSpec((_BM, _F), lambda i: (i, 0)),
        out_shape=jax.ShapeDtypeStruct((_N, _F), jnp.float32),
    )(agg_parts, deg3, b2)


def kernel(x, edge_index, W, b):
    edges = edge_index.reshape(2, _NCHUNK, _CHUNK)
    deg_parts = _deg_kernel(edges)
    deg3 = deg_parts.reshape(_NC, _NPAD // 128, 128)
    hs = _mm_call(x, W, deg3)
    agg_parts = _agg_kernel(hs, edges)
    out = _fin_call(agg_parts, deg3, b.reshape(1, _F))
    return out
